# baseline (speedup 1.0000x reference)
import functools

import jax
import jax.numpy as jnp
from jax import lax
from jax.experimental import pallas as pl
from jax.experimental.pallas import tpu as pltpu
from jax.experimental.pallas import tpu_sc as plsc

NC = 2
NS = 16
NW = NC * NS
BATCH = 125


K = 5


def _deg_body(nb, rpt, dstb, ones_hbm, zeros_hbm, out_hbm, dstv, ones_v, acc,
              *sems):
    cid = lax.axis_index("c")
    sid = lax.axis_index("s")
    wid = cid * NS + sid
    ngroups = nb // K
    pltpu.sync_copy(zeros_hbm.at[pl.ds(sid * rpt, rpt)],
                    acc.at[pl.ds(sid * rpt, rpt)])
    pltpu.sync_copy(dstb.at[pl.ds(wid * nb, nb)], dstv)
    pltpu.sync_copy(ones_hbm, ones_v)
    plsc.subcore_barrier()

    def s_start(j, b):
        pltpu.async_copy(ones_v, acc.at[dstv.at[j]], sems[b], add=True)

    def s_wait(b):
        pltpu.make_async_copy(ones_v, acc.at[dstv.at[0]], sems[b]).wait()

    for b in range(K):
        s_start(b, b)

    def group(gi, c):
        j0 = gi * K
        for b in range(K):
            s_wait(b)
            s_start(j0 + b, b)
        return c

    lax.fori_loop(1, ngroups, group, 0)
    for b in range(K):
        s_wait(b)
    plsc.subcore_barrier()
    pltpu.sync_copy(acc.at[pl.ds(sid * rpt, rpt)],
                    out_hbm.at[cid].at[pl.ds(sid * rpt, rpt)])


def _agg_body(nb, rpt, g_hbm, srcb, dstb, zeros_hbm, out_hbm,
              srcv, dstv, rows, acc, g_sh, *sems):
    cid = lax.axis_index("c")
    sid = lax.axis_index("s")
    wid = cid * NS + sid
    ngroups = nb // K
    n_g = g_hbm.shape[0]
    gpt = n_g // NS
    pltpu.sync_copy(zeros_hbm.at[pl.ds(sid * rpt, rpt)],
                    acc.at[pl.ds(sid * rpt, rpt)])
    pltpu.sync_copy(g_hbm.at[pl.ds(sid * gpt, gpt)],
                    g_sh.at[pl.ds(sid * gpt, gpt)])
    pltpu.sync_copy(srcb.at[pl.ds(wid * nb, nb)], srcv)
    pltpu.sync_copy(dstb.at[pl.ds(wid * nb, nb)], dstv)
    plsc.subcore_barrier()

    def g_start(j, b):
        pltpu.async_copy(g_sh.at[srcv.at[j]], rows.at[b], sems[b])

    def g_wait(j, b):
        pltpu.make_async_copy(g_sh.at[srcv.at[j]], rows.at[b], sems[b]).wait()

    def s_start(j, b):
        pltpu.async_copy(rows.at[b], acc.at[dstv.at[j]], sems[K + b], add=True)

    def s_wait(j, b):
        pltpu.make_async_copy(rows.at[b], acc.at[dstv.at[j]],
                              sems[K + b]).wait()

    for b in range(K - 1):
        g_start(b, b)
    for b in range(K):
        g_wait(b, b)
        s_start(b, b)
        if b >= 1:
            s_wait(b, (b - 1) % K)
        g_start(b + K - 1, (b - 1) % K)

    def group(gi, c):
        j0 = gi * K
        for b in range(K):
            j = j0 + b
            g_wait(j, b)
            s_start(j, b)
            s_wait(j, (b - 1) % K)
            g_start(j + K - 1, (b - 1) % K)
        return c

    lax.fori_loop(1, ngroups - 1, group, 0)
    j0 = (ngroups - 1) * K
    for b in range(K):
        j = j0 + b
        g_wait(j, b)
        s_start(j, b)
        s_wait(j, (b - 1) % K)
        if j + K - 1 < nb:
            g_start(j + K - 1, (b - 1) % K)
    s_wait(nb - 1, (K - 1) % K)
    plsc.subcore_barrier()
    pltpu.sync_copy(acc.at[pl.ds(sid * rpt, rpt)],
                    out_hbm.at[cid].at[pl.ds(sid * rpt, rpt)])


@functools.lru_cache(maxsize=None)
def _make_deg(n_acc, nb):
    rpt = n_acc // NS
    return pl.kernel(
        functools.partial(_deg_body, nb, rpt),
        out_type=jax.ShapeDtypeStruct((NC, n_acc, 32), jnp.float32),
        mesh=plsc.VectorSubcoreMesh(core_axis_name="c", subcore_axis_name="s"),
        scratch_types=[
            pltpu.VMEM((nb, BATCH), jnp.int32),
            pltpu.VMEM((BATCH, 32), jnp.float32),
            pltpu.VMEM_SHARED((n_acc, 32), jnp.float32),
            *([pltpu.SemaphoreType.DMA] * K),
        ],
        compiler_params=pltpu.CompilerParams(use_tc_tiling_on_sc=False),
    )


@functools.lru_cache(maxsize=None)
def _make_agg(n_acc, nb, d, n_g):
    rpt = n_acc // NS
    return pl.kernel(
        functools.partial(_agg_body, nb, rpt),
        out_type=jax.ShapeDtypeStruct((NC, n_acc, d), jnp.float32),
        mesh=plsc.VectorSubcoreMesh(core_axis_name="c", subcore_axis_name="s"),
        scratch_types=[
            pltpu.VMEM((nb, BATCH), jnp.int32),
            pltpu.VMEM((nb, BATCH), jnp.int32),
            pltpu.VMEM((K, BATCH, d), jnp.float32),
            pltpu.VMEM_SHARED((n_acc, d), jnp.float32),
            pltpu.VMEM_SHARED((n_g, d), jnp.float32),
            *([pltpu.SemaphoreType.DMA] * (2 * K)),
        ],
        compiler_params=pltpu.CompilerParams(use_tc_tiling_on_sc=False),
    )


def _tc_first_body(x_ref, w_ref, h_ref):
    h_ref[...] = jnp.dot(x_ref[...], w_ref[...],
                         preferred_element_type=jnp.float32)


def _tc_scale_body(degp_ref, h_ref, dinv_ref, g_ref):
    deg = degp_ref[0] + degp_ref[1] + 1.0
    dinv = lax.rsqrt(deg)
    dinv_ref[...] = dinv
    g_ref[...] = h_ref[...] * dinv


def _tc_mid_body(p_ref, g_ref, dinv_ref, b_ref, w_ref, gout_ref):
    dinv = dinv_ref[...]
    t = jnp.maximum((p_ref[0] + p_ref[1] + g_ref[...]) * dinv + b_ref[...], 0.0)
    gout_ref[...] = jnp.dot(t, w_ref[...],
                            preferred_element_type=jnp.float32) * dinv


def _tc_fin_body(p_ref, g_ref, dinv_ref, b_ref, wl_ref, bl_ref, out_ref):
    dinv = dinv_ref[...]
    t = jnp.maximum((p_ref[0] + p_ref[1] + g_ref[...]) * dinv + b_ref[...], 0.0)
    out_ref[...] = jnp.dot(t, wl_ref[...],
                           preferred_element_type=jnp.float32) + bl_ref[...]


def _tc_first(x, w, n_g):
    n, kk = x.shape
    d = w.shape[1]
    bn = n_g // 4
    return pl.pallas_call(
        _tc_first_body,
        grid=(4,),
        in_specs=[
            pl.BlockSpec((bn, kk), lambda i: (i, 0)),
            pl.BlockSpec((kk, d), lambda i: (0, 0)),
        ],
        out_specs=pl.BlockSpec((bn, d), lambda i: (i, 0)),
        out_shape=jax.ShapeDtypeStruct((n_g, d), jnp.float32),
    )(x, w)


def _tc_scale(degp, hp):
    rp = hp.shape[0]
    return pl.pallas_call(
        _tc_scale_body,
        grid=(1,),
        in_specs=[
            pl.BlockSpec((NC, rp, 128), lambda i: (0, 0, 0)),
            pl.BlockSpec((rp, 128), lambda i: (0, 0)),
        ],
        out_specs=[
            pl.BlockSpec((rp, 128), lambda i: (0, 0)),
            pl.BlockSpec((rp, 128), lambda i: (0, 0)),
        ],
        out_shape=[
            jax.ShapeDtypeStruct((rp, 128), jnp.float32),
            jax.ShapeDtypeStruct((rp, 128), jnp.float32),
        ],
    )(degp, hp)


def _tc_mid(p, g, dinv, b, w):
    rp = g.shape[0]
    return pl.pallas_call(
        _tc_mid_body,
        grid=(1,),
        in_specs=[
            pl.BlockSpec((NC, rp, 128), lambda i: (0, 0, 0)),
            pl.BlockSpec((rp, 128), lambda i: (0, 0)),
            pl.BlockSpec((rp, 128), lambda i: (0, 0)),
            pl.BlockSpec((1, 128), lambda i: (0, 0)),
            pl.BlockSpec((128, 128), lambda i: (0, 0)),
        ],
        out_specs=pl.BlockSpec((rp, 128), lambda i: (0, 0)),
        out_shape=jax.ShapeDtypeStruct((rp, 128), jnp.float32),
    )(p, g, dinv, b, w)


def _tc_fin(p, g, dinv, b, wl, bl):
    rp = g.shape[0]
    return pl.pallas_call(
        _tc_fin_body,
        grid=(1,),
        in_specs=[
            pl.BlockSpec((NC, rp, 128), lambda i: (0, 0, 0)),
            pl.BlockSpec((rp, 128), lambda i: (0, 0)),
            pl.BlockSpec((rp, 128), lambda i: (0, 0)),
            pl.BlockSpec((1, 128), lambda i: (0, 0)),
            pl.BlockSpec((128, 128), lambda i: (0, 0)),
            pl.BlockSpec((1, 128), lambda i: (0, 0)),
        ],
        out_specs=pl.BlockSpec((rp, 128), lambda i: (0, 0)),
        out_shape=jax.ShapeDtypeStruct((rp, 128), jnp.float32),
    )(p, g, dinv, b, wl, bl)


def _pad2(a, r, c):
    return jnp.pad(a, ((0, r - a.shape[0]), (0, c - a.shape[1])))


def _blk4(w):
    return jnp.kron(jnp.eye(4, dtype=jnp.float32), _pad2(w, 32, 32))


def _tile4(b, d):
    return jnp.tile(jnp.pad(b, (0, 32 - d)), 4)[None, :]


def kernel(x, edge_index, W1, b1, W2, b2, W3, b3, W4, b4, Wl, bl):
    n, k = x.shape
    e = edge_index.shape[1]

    chunk = NW * BATCH
    e_pad = -(-e // chunk) * chunk
    nb = e_pad // chunk
    n_acc = -(-(n + 1) // (NS * 8)) * (NS * 8)
    n_g = -(-n // 32) * 32
    rp = n_g // 4
    rp_acc = n_acc // 4

    assert e_pad == e, "edge count must divide NW*BATCH evenly"
    srcp = edge_index[0].reshape(e // BATCH, BATCH)
    dstp = edge_index[1].reshape(e // BATCH, BATCH)

    zeros32 = jnp.zeros((n_acc, 32), jnp.float32)
    ones32 = jnp.ones((BATCH, 32), jnp.float32)

    w1p = _pad2(W1, k, 32)
    w2blk = _blk4(W2)
    w3blk = _blk4(W3)
    w4blk = _blk4(W4)
    wlblk = jnp.kron(jnp.eye(4, dtype=jnp.float32), _pad2(Wl, 32, 32))
    b1p = _tile4(b1, 20)
    b2p = _tile4(b2, 25)
    b3p = _tile4(b3, 20)
    b4p = _tile4(b4, 10)
    blp = _tile4(bl, 3)

    deg = _make_deg(n_acc, nb)(dstp, ones32, zeros32)
    degp = deg.reshape(NC, rp_acc, 128)

    agg = _make_agg(n_acc, nb, 32, n_g)

    def packed(part):
        return part.reshape(NC, rp_acc, 128)

    h1p = _tc_first(x, w1p, n_g).reshape(rp, 128)
    dinv, g1p = _tc_scale(degp, h1p)
    p1 = agg(g1p.reshape(n_g, 32), srcp, dstp, zeros32)
    g2p = _tc_mid(packed(p1), g1p, dinv, b1p, w2blk)
    p2 = agg(g2p.reshape(n_g, 32), srcp, dstp, zeros32)
    g3p = _tc_mid(packed(p2), g2p, dinv, b2p, w3blk)
    p3 = agg(g3p.reshape(n_g, 32), srcp, dstp, zeros32)
    g4p = _tc_mid(packed(p3), g3p, dinv, b3p, w4blk)
    p4 = agg(g4p.reshape(n_g, 32), srcp, dstp, zeros32)
    outp = _tc_fin(packed(p4), g4p, dinv, b4p, wlblk, blp)
    return outp.reshape(n_g, 32)[:n, :3]


# speedup vs baseline: 1.0428x; 1.0428x over previous
import functools

import jax
import jax.numpy as jnp
from jax import lax
from jax.experimental import pallas as pl
from jax.experimental.pallas import tpu as pltpu
from jax.experimental.pallas import tpu_sc as plsc

NC = 2
NS = 16
NW = NC * NS
BATCH = 125


K = 5


def _deg_body(nb, rpt, ehalf, edgeb, ones_hbm, zeros_hbm, out_hbm, dstv,
              ones_v, acc, *sems):
    cid = lax.axis_index("c")
    sid = lax.axis_index("s")
    wid = cid * NS + sid
    ngroups = nb // K
    pltpu.sync_copy(zeros_hbm.at[pl.ds(sid * rpt, rpt)],
                    acc.at[pl.ds(sid * rpt, rpt)])
    pltpu.sync_copy(edgeb.at[pl.ds(ehalf + wid * nb, nb)], dstv)
    pltpu.sync_copy(ones_hbm, ones_v)
    plsc.subcore_barrier()

    def s_start(j, b):
        pltpu.async_copy(ones_v, acc.at[dstv.at[j]], sems[b], add=True)

    def s_wait(b):
        pltpu.make_async_copy(ones_v, acc.at[dstv.at[0]], sems[b]).wait()

    for b in range(K):
        s_start(b, b)

    def group(gi, c):
        j0 = gi * K
        for b in range(K):
            s_wait(b)
            s_start(j0 + b, b)
        return c

    lax.fori_loop(1, ngroups, group, 0)
    for b in range(K):
        s_wait(b)
    plsc.subcore_barrier()
    pltpu.sync_copy(acc.at[pl.ds(sid * rpt, rpt)],
                    out_hbm.at[cid].at[pl.ds(sid * rpt, rpt)])


def _agg_body(nb, rpt, ehalf, g_hbm, edgeb, zeros_hbm, out_hbm,
              srcv, dstv, rows, acc, g_sh, *sems):
    cid = lax.axis_index("c")
    sid = lax.axis_index("s")
    wid = cid * NS + sid
    ngroups = nb // K
    n_g = g_hbm.shape[0]
    gpt = n_g // NS
    pltpu.sync_copy(zeros_hbm.at[pl.ds(sid * rpt, rpt)],
                    acc.at[pl.ds(sid * rpt, rpt)])
    pltpu.sync_copy(g_hbm.at[pl.ds(sid * gpt, gpt)],
                    g_sh.at[pl.ds(sid * gpt, gpt)])
    pltpu.sync_copy(edgeb.at[pl.ds(wid * nb, nb)], srcv)
    pltpu.sync_copy(edgeb.at[pl.ds(ehalf + wid * nb, nb)], dstv)
    plsc.subcore_barrier()

    def g_start(j, b):
        pltpu.async_copy(g_sh.at[srcv.at[j]], rows.at[b], sems[b])

    def g_wait(j, b):
        pltpu.make_async_copy(g_sh.at[srcv.at[j]], rows.at[b], sems[b]).wait()

    def s_start(j, b):
        pltpu.async_copy(rows.at[b], acc.at[dstv.at[j]], sems[K + b], add=True)

    def s_wait(j, b):
        pltpu.make_async_copy(rows.at[b], acc.at[dstv.at[j]],
                              sems[K + b]).wait()

    for b in range(K - 1):
        g_start(b, b)
    for b in range(K):
        g_wait(b, b)
        s_start(b, b)
        if b >= 1:
            s_wait(b, (b - 1) % K)
        g_start(b + K - 1, (b - 1) % K)

    def group(gi, c):
        j0 = gi * K
        for b in range(K):
            j = j0 + b
            g_wait(j, b)
            s_start(j, b)
            s_wait(j, (b - 1) % K)
            g_start(j + K - 1, (b - 1) % K)
        return c

    lax.fori_loop(1, ngroups - 1, group, 0)
    j0 = (ngroups - 1) * K
    for b in range(K):
        j = j0 + b
        g_wait(j, b)
        s_start(j, b)
        s_wait(j, (b - 1) % K)
        if j + K - 1 < nb:
            g_start(j + K - 1, (b - 1) % K)
    s_wait(nb - 1, (K - 1) % K)
    plsc.subcore_barrier()
    pltpu.sync_copy(acc.at[pl.ds(sid * rpt, rpt)],
                    out_hbm.at[cid].at[pl.ds(sid * rpt, rpt)])


@functools.lru_cache(maxsize=None)
def _make_deg(n_acc, nb, ehalf):
    rpt = n_acc // NS
    return pl.kernel(
        functools.partial(_deg_body, nb, rpt, ehalf),
        out_type=jax.ShapeDtypeStruct((NC, n_acc, 32), jnp.float32),
        mesh=plsc.VectorSubcoreMesh(core_axis_name="c", subcore_axis_name="s"),
        scratch_types=[
            pltpu.VMEM((nb, BATCH), jnp.int32),
            pltpu.VMEM((BATCH, 32), jnp.float32),
            pltpu.VMEM_SHARED((n_acc, 32), jnp.float32),
            *([pltpu.SemaphoreType.DMA] * K),
        ],
        compiler_params=pltpu.CompilerParams(use_tc_tiling_on_sc=False),
    )


@functools.lru_cache(maxsize=None)
def _make_agg(n_acc, nb, d, n_g, ehalf):
    rpt = n_acc // NS
    return pl.kernel(
        functools.partial(_agg_body, nb, rpt, ehalf),
        out_type=jax.ShapeDtypeStruct((NC, n_acc, d), jnp.float32),
        mesh=plsc.VectorSubcoreMesh(core_axis_name="c", subcore_axis_name="s"),
        scratch_types=[
            pltpu.VMEM((nb, BATCH), jnp.int32),
            pltpu.VMEM((nb, BATCH), jnp.int32),
            pltpu.VMEM((K, BATCH, d), jnp.float32),
            pltpu.VMEM_SHARED((n_acc, d), jnp.float32),
            pltpu.VMEM_SHARED((n_g, d), jnp.float32),
            *([pltpu.SemaphoreType.DMA] * (2 * K)),
        ],
        compiler_params=pltpu.CompilerParams(use_tc_tiling_on_sc=False),
    )


def _tc_first_body(n, x_ref, w_ref, h_ref):
    h = jnp.dot(x_ref[...], w_ref[...], preferred_element_type=jnp.float32)
    h_ref[pl.ds(0, n), :] = h
    h_ref[pl.ds(n, h_ref.shape[0] - n), :] = jnp.zeros(
        (h_ref.shape[0] - n, h_ref.shape[1]), jnp.float32)


def _tc_scale_body(degp_ref, h_ref, dinv_ref, g_ref):
    deg = degp_ref[0] + degp_ref[1] + 1.0
    dinv = lax.rsqrt(deg)
    dinv_ref[...] = dinv
    g_ref[...] = h_ref[...] * dinv


def _tc_mid_body(p_ref, g_ref, dinv_ref, b_ref, w_ref, gout_ref):
    dinv = dinv_ref[...]
    b4 = jnp.concatenate([b_ref[...]] * 4, axis=1)
    t = jnp.maximum((p_ref[0] + p_ref[1] + g_ref[...]) * dinv + b4, 0.0)
    gout_ref[...] = jnp.dot(t, w_ref[...],
                            preferred_element_type=jnp.float32) * dinv


def _tc_fin_body(p_ref, g_ref, dinv_ref, b_ref, wl_ref, bl_ref, out_ref):
    dinv = dinv_ref[...]
    b4 = jnp.concatenate([b_ref[...]] * 4, axis=1)
    bl4 = jnp.concatenate([bl_ref[...]] * 4, axis=1)
    t = jnp.maximum((p_ref[0] + p_ref[1] + g_ref[...]) * dinv + b4, 0.0)
    out_ref[...] = jnp.dot(t, wl_ref[...],
                           preferred_element_type=jnp.float32) + bl4


def _tc_first(x, w, n_g):
    n, kk = x.shape
    d = w.shape[1]
    return pl.pallas_call(
        functools.partial(_tc_first_body, n),
        grid=(1,),
        in_specs=[
            pl.BlockSpec((n, kk), lambda i: (0, 0)),
            pl.BlockSpec((kk, d), lambda i: (0, 0)),
        ],
        out_specs=pl.BlockSpec((n_g, d), lambda i: (0, 0)),
        out_shape=jax.ShapeDtypeStruct((n_g, d), jnp.float32),
    )(x, w)


def _tc_scale(degp, hp):
    rp = hp.shape[0]
    return pl.pallas_call(
        _tc_scale_body,
        grid=(1,),
        in_specs=[
            pl.BlockSpec((NC, rp, 128), lambda i: (0, 0, 0)),
            pl.BlockSpec((rp, 128), lambda i: (0, 0)),
        ],
        out_specs=[
            pl.BlockSpec((rp, 128), lambda i: (0, 0)),
            pl.BlockSpec((rp, 128), lambda i: (0, 0)),
        ],
        out_shape=[
            jax.ShapeDtypeStruct((rp, 128), jnp.float32),
            jax.ShapeDtypeStruct((rp, 128), jnp.float32),
        ],
    )(degp, hp)


def _tc_mid(p, g, dinv, b, w):
    rp = g.shape[0]
    return pl.pallas_call(
        _tc_mid_body,
        grid=(1,),
        in_specs=[
            pl.BlockSpec((NC, rp, 128), lambda i: (0, 0, 0)),
            pl.BlockSpec((rp, 128), lambda i: (0, 0)),
            pl.BlockSpec((rp, 128), lambda i: (0, 0)),
            pl.BlockSpec((1, 32), lambda i: (0, 0)),
            pl.BlockSpec((128, 128), lambda i: (0, 0)),
        ],
        out_specs=pl.BlockSpec((rp, 128), lambda i: (0, 0)),
        out_shape=jax.ShapeDtypeStruct((rp, 128), jnp.float32),
    )(p, g, dinv, b, w)


def _tc_fin(p, g, dinv, b, wl, bl):
    rp = g.shape[0]
    return pl.pallas_call(
        _tc_fin_body,
        grid=(1,),
        in_specs=[
            pl.BlockSpec((NC, rp, 128), lambda i: (0, 0, 0)),
            pl.BlockSpec((rp, 128), lambda i: (0, 0)),
            pl.BlockSpec((rp, 128), lambda i: (0, 0)),
            pl.BlockSpec((1, 32), lambda i: (0, 0)),
            pl.BlockSpec((128, 128), lambda i: (0, 0)),
            pl.BlockSpec((1, 32), lambda i: (0, 0)),
        ],
        out_specs=pl.BlockSpec((rp, 128), lambda i: (0, 0)),
        out_shape=jax.ShapeDtypeStruct((rp, 128), jnp.float32),
    )(p, g, dinv, b, wl, bl)


def _pad2(a, r, c):
    return jnp.pad(a, ((0, r - a.shape[0]), (0, c - a.shape[1])))


def _blk4(w):
    return jnp.kron(jnp.eye(4, dtype=jnp.float32), _pad2(w, 32, 32))


def _tile4(b, d):
    return jnp.tile(jnp.pad(b, (0, 32 - d)), 4)[None, :]


def kernel(x, edge_index, W1, b1, W2, b2, W3, b3, W4, b4, Wl, bl):
    n, k = x.shape
    e = edge_index.shape[1]

    chunk = NW * BATCH
    e_pad = -(-e // chunk) * chunk
    nb = e_pad // chunk
    n_acc = -(-(n + 1) // (NS * 8)) * (NS * 8)
    n_g = -(-n // 32) * 32
    rp = n_g // 4
    rp_acc = n_acc // 4

    assert e_pad == e, "edge count must divide NW*BATCH evenly"
    edgeb = edge_index.reshape(2 * (e // BATCH), BATCH)

    zeros32 = jnp.zeros((n_acc, 32), jnp.float32)
    ones32 = jnp.ones((BATCH, 32), jnp.float32)

    w1p = _pad2(W1, k, 32)
    w2blk = _blk4(W2)
    w3blk = _blk4(W3)
    w4blk = _blk4(W4)
    wlblk = jnp.kron(jnp.eye(4, dtype=jnp.float32), _pad2(Wl, 32, 32))
    b1p = jnp.pad(b1, (0, 12))[None, :]
    b2p = jnp.pad(b2, (0, 7))[None, :]
    b3p = jnp.pad(b3, (0, 12))[None, :]
    b4p = jnp.pad(b4, (0, 22))[None, :]
    blp = jnp.pad(bl, (0, 29))[None, :]

    deg = _make_deg(n_acc, nb, e // BATCH)(edgeb, ones32, zeros32)
    degp = deg.reshape(NC, rp_acc, 128)

    agg = _make_agg(n_acc, nb, 32, n_g, e // BATCH)

    def packed(part):
        return part.reshape(NC, rp_acc, 128)

    h1p = _tc_first(x, w1p, n_g).reshape(rp, 128)
    dinv, g1p = _tc_scale(degp, h1p)
    p1 = agg(g1p.reshape(n_g, 32), edgeb, zeros32)
    g2p = _tc_mid(packed(p1), g1p, dinv, b1p, w2blk)
    p2 = agg(g2p.reshape(n_g, 32), edgeb, zeros32)
    g3p = _tc_mid(packed(p2), g2p, dinv, b2p, w3blk)
    p3 = agg(g3p.reshape(n_g, 32), edgeb, zeros32)
    g4p = _tc_mid(packed(p3), g3p, dinv, b3p, w4blk)
    p4 = agg(g4p.reshape(n_g, 32), edgeb, zeros32)
    outp = _tc_fin(packed(p4), g4p, dinv, b4p, wlblk, blp)
    return outp.reshape(n_g, 32)[:n, :3]
